# Initial kernel scaffold; baseline (speedup 1.0000x reference)
import jax
import jax.numpy as jnp
from jax.experimental import pallas as pl


def kernel(x, edge_index_0, edge_index_1, edge_index_2, W0, b0, aw1_0, ab1_0, aw2_0, W1, b1, aw1_1, ab1_1, aw2_1, W2, b2, aw1_2, ab1_2, aw2_2):
    raise NotImplementedError("write your pallas kernel here")


# speedup vs baseline: 2.9483x; 2.9483x over previous
import functools

import jax
import jax.numpy as jnp
from jax import lax
from jax.experimental import pallas as pl
from jax.experimental.pallas import tpu as pltpu
from jax.experimental.pallas import tpu_sc as plsc

N = 10000
E = 160000
D = 128
DQ = 64

NC = 2
NS = 16
NW = NC * NS
EPT = E // NW
CH = 128
KC = (EPT + CH - 1) // CH
EPT_PAD = KC * CH
NPAD = 10112
RPT = NPAD // NS
DW = 16


def _sc_body(x_hbm, srcp_hbm, dstp_hbm, zrows_hbm, ones_hbm,
             agg_hbm, deg_hbm,
             src_idx_v, dst_idx_v, rows_v, ones_v, acc_sh, sem):
    c = lax.axis_index("c")
    s = lax.axis_index("s")
    w = c * NS + s
    r0 = s * RPT

    pltpu.sync_copy(ones_hbm, ones_v)

    for e in range(3):
        pltpu.sync_copy(zrows_hbm, acc_sh.at[pl.ds(r0, RPT)])
        pltpu.sync_copy(dstp_hbm.at[e, w], dst_idx_v)
        plsc.subcore_barrier()

        def dchunk(k, carry):
            pltpu.sync_copy(ones_v, acc_sh.at[dst_idx_v.at[k]], add=True)
            return carry

        lax.fori_loop(0, KC, dchunk, 0)
        plsc.subcore_barrier()
        pltpu.sync_copy(acc_sh.at[pl.ds(r0, RPT)],
                        deg_hbm.at[e, c, pl.ds(r0, RPT)])

        pltpu.sync_copy(zrows_hbm, acc_sh.at[pl.ds(r0, RPT)])
        pltpu.sync_copy(srcp_hbm.at[e, w], src_idx_v)
        plsc.subcore_barrier()

        def gchunk(k, carry):
            pltpu.async_copy(x_hbm.at[src_idx_v.at[k]], rows_v, sem).wait()
            pltpu.sync_copy(rows_v, acc_sh.at[dst_idx_v.at[k]], add=True)
            return carry

        lax.fori_loop(0, KC, gchunk, 0)
        plsc.subcore_barrier()
        pltpu.sync_copy(acc_sh.at[pl.ds(r0, RPT)],
                        agg_hbm.at[e, c, pl.ds(r0, RPT)])


def _sc_segment_sums(x, srcp, dstp):
    zrows = jnp.zeros((RPT, D), jnp.float32)
    ones = jnp.ones((CH, D), jnp.float32)
    mesh = plsc.VectorSubcoreMesh(core_axis_name="c", subcore_axis_name="s")
    fn = functools.partial(
        pl.kernel,
        out_type=(jax.ShapeDtypeStruct((3, NC, NPAD, D), jnp.float32),
                  jax.ShapeDtypeStruct((3, NC, NPAD, D), jnp.float32)),
        mesh=mesh,
        scratch_types=[
            pltpu.VMEM((KC, CH), jnp.int32),
            pltpu.VMEM((KC, CH), jnp.int32),
            pltpu.VMEM((CH, D), jnp.float32),
            pltpu.VMEM((CH, D), jnp.float32),
            pltpu.VMEM_SHARED((NPAD, D), jnp.float32),
            pltpu.SemaphoreType.DMA,
        ],
    )(_sc_body)
    return fn(x, srcp, dstp, zrows, ones)


def _tc_body(agg_ref, deg_ref, w_ref, b_ref, a1_ref, ab1_ref, a2_ref,
             out_ref, attn_ref):
    ds = []
    ss = []
    for e in range(3):
        agge = agg_ref[e, 0] + agg_ref[e, 1]
        dege = deg_ref[e, 0, :, 0] + deg_ref[e, 1, :, 0]
        m = agge / jnp.maximum(dege, 1.0)[:, None]
        d_e = jnp.dot(m, w_ref[e], preferred_element_type=jnp.float32)
        d_e = d_e + b_ref[e][None, :]
        h = jnp.tanh(jnp.dot(d_e, a1_ref[e],
                             preferred_element_type=jnp.float32)
                     + ab1_ref[e][None, :])
        s_e = jnp.sum(h * a2_ref[e][None, :], axis=1)
        ds.append(d_e)
        ss.append(s_e)
    s = jnp.stack(ss, axis=0)
    mx = jnp.max(s, axis=0, keepdims=True)
    ex = jnp.exp(s - mx)
    att = ex / jnp.sum(ex, axis=0, keepdims=True)
    out_ref[...] = (att[0][:, None] * ds[0] + att[1][:, None] * ds[1]
                    + att[2][:, None] * ds[2])
    attn_ref[...] = att[:, :, None]


def _tc_combine(agg, deg, wst, bst, a1st, ab1st, a2st):
    B = 1000
    grid = (N // B,)
    return pl.pallas_call(
        _tc_body,
        grid=grid,
        in_specs=[
            pl.BlockSpec((3, NC, B, D), lambda i: (0, 0, i, 0)),
            pl.BlockSpec((3, NC, B, D), lambda i: (0, 0, i, 0)),
            pl.BlockSpec((3, D, D), lambda i: (0, 0, 0)),
            pl.BlockSpec((3, D), lambda i: (0, 0)),
            pl.BlockSpec((3, D, DQ), lambda i: (0, 0, 0)),
            pl.BlockSpec((3, DQ), lambda i: (0, 0)),
            pl.BlockSpec((3, DQ), lambda i: (0, 0)),
        ],
        out_specs=[
            pl.BlockSpec((B, D), lambda i: (i, 0)),
            pl.BlockSpec((3, B, 1), lambda i: (0, i, 0)),
        ],
        out_shape=[
            jax.ShapeDtypeStruct((N, D), jnp.float32),
            jax.ShapeDtypeStruct((3, N, 1), jnp.float32),
        ],
    )(agg, deg, wst, bst, a1st, ab1st, a2st)


def kernel(x, edge_index_0, edge_index_1, edge_index_2,
           W0, b0, aw1_0, ab1_0, aw2_0,
           W1, b1, aw1_1, ab1_1, aw2_1,
           W2, b2, aw1_2, ab1_2, aw2_2):
    srcs = []
    dsts = []
    pad = EPT_PAD - EPT
    for ei in (edge_index_0, edge_index_1, edge_index_2):
        src = ei[0].reshape(NW, EPT)
        dst = ei[1].reshape(NW, EPT)
        src = jnp.pad(src, ((0, 0), (0, pad)))
        dst = jnp.pad(dst, ((0, 0), (0, pad)), constant_values=N)
        srcs.append(src.reshape(NW, KC, CH))
        dsts.append(dst.reshape(NW, KC, CH))
    srcp = jnp.stack(srcs, axis=0)
    dstp = jnp.stack(dsts, axis=0)

    agg, deg = _sc_segment_sums(x, srcp, dstp)

    wst = jnp.stack([W0, W1, W2], axis=0)
    bst = jnp.stack([b0, b1, b2], axis=0)
    a1st = jnp.stack([aw1_0, aw1_1, aw1_2], axis=0)
    ab1st = jnp.stack([ab1_0, ab1_1, ab1_2], axis=0)
    a2st = jnp.stack([aw2_0[:, 0], aw2_1[:, 0], aw2_2[:, 0]], axis=0)

    out_embs, attn = _tc_combine(agg, deg, wst, bst, a1st, ab1st, a2st)
    return out_embs, attn
